# baseline (speedup 1.0000x reference)
import functools

import jax
import jax.numpy as jnp
from jax import lax
from jax.experimental import pallas as pl
from jax.experimental.pallas import tpu as pltpu
from jax.experimental.pallas import tpu_sc as plsc

T, N, E, D, H, L = 4, 10000, 320000, 128, 128, 2

NC, NS = 2, 16
NW = NC * NS
CH = 128
NCHUNK = 80
EPW = NCHUNK * CH
NP = 10240
RPT = NP // NS

BN = 1000


NH = 1
NCH_H = NCHUNK // NH
EPH = NCH_H * CH


def _mp_body(y_hbm, src_hbm, dst_hbm, w_hbm, zeros_hbm, out_hbm,
             src_v, dst_v, w_v, rows0, rows1, acc_sh, gs0, gs1):
    c = lax.axis_index("c")
    s = lax.axis_index("s")
    wid = s * NC + c

    pltpu.sync_copy(zeros_hbm, acc_sh.at[pl.ds(s * RPT, RPT)])
    plsc.subcore_barrier()

    for half in range(NH):
        pltpu.sync_copy(src_hbm.at[wid, half], src_v)
        pltpu.sync_copy(dst_hbm.at[wid, half], dst_v)
        pltpu.sync_copy(w_hbm.at[wid, half], w_v)
        def chunk(j, carry):
            pltpu.async_copy(y_hbm.at[src_v.at[j]], rows0, gs0).wait()

            def _scale(e, carry2):
                wspl = plsc.load_gather(
                    w_v, [jnp.full((16,), j * CH + e, jnp.int32)])
                for v in range(8):
                    rows0[e, pl.ds(v * 16, 16)] = (
                        rows0[e, pl.ds(v * 16, 16)] * wspl)
                return carry2

            lax.fori_loop(0, CH, _scale, 0)
            pltpu.sync_copy(rows0, acc_sh.at[dst_v.at[j]], add=True)
            return carry

        lax.fori_loop(0, NCH_H, chunk, 0)

    plsc.subcore_barrier()
    pltpu.sync_copy(acc_sh.at[pl.ds(s * RPT, RPT)],
                    out_hbm.at[c].at[pl.ds(s * RPT, RPT)])


def _mp_call(y, src_p, dst_p, w_p, zeros):
    mesh = plsc.VectorSubcoreMesh(core_axis_name="c", subcore_axis_name="s")
    return pl.kernel(
        _mp_body,
        out_type=jax.ShapeDtypeStruct((NC, NP, H), jnp.float32),
        mesh=mesh,
        compiler_params=pltpu.CompilerParams(needs_layout_passes=False),
        scratch_types=[
            pltpu.VMEM((NCH_H + 1, CH), jnp.int32),
            pltpu.VMEM((NCH_H, CH), jnp.int32),
            pltpu.VMEM((EPH,), jnp.float32),
            pltpu.VMEM((CH, H), jnp.float32),
            pltpu.VMEM((CH, H), jnp.float32),
            pltpu.VMEM_SHARED((NP, H), jnp.float32),
            pltpu.SemaphoreType.DMA,
            pltpu.SemaphoreType.DMA,
        ],
    )(y, src_p, dst_p, w_p, zeros)


def _gi_body(x_ref, w_ref, b_ref, o_ref):
    o_ref[...] = lax.dot_general(
        x_ref[...], w_ref[...], (((1,), (1,)), ((), ())),
        preferred_element_type=jnp.float32) + b_ref[...]


def _rsqrt_body(a0_ref, a1_ref, o_ref):
    o_ref[...] = lax.rsqrt(1.0 + a0_ref[...] + a1_ref[...])


def _gru_body(gi_ref, h_ref, whh_ref, bhh_ref, w0_ref, dinv_ref, y0_ref):
    h = h_ref[...]
    gh = lax.dot_general(h, whh_ref[...], (((1,), (1,)), ((), ())),
                         preferred_element_type=jnp.float32) + bhh_ref[...]
    gi = gi_ref[...]
    r = jax.nn.sigmoid(gi[:, :H] + gh[:, :H])
    z = jax.nn.sigmoid(gi[:, H:2 * H] + gh[:, H:2 * H])
    nc = jnp.tanh(gi[:, 2 * H:] + r * gh[:, 2 * H:])
    hn = (1.0 - z) * nc + z * h
    y0_ref[...] = lax.dot_general(
        hn, w0_ref[...], (((1,), (0,)), ((), ())),
        preferred_element_type=jnp.float32) * dinv_ref[...]


def _post_body(a0_ref, a1_ref, y_ref, dinv_ref, b_ref, w_ref, o_ref):
    dinv = dinv_ref[...]
    h1 = jax.nn.relu((a0_ref[...] + a1_ref[...] + y_ref[...]) * dinv + b_ref[...])
    o_ref[...] = lax.dot_general(
        h1, w_ref[...], (((1,), (0,)), ((), ())),
        preferred_element_type=jnp.float32) * dinv


def _final_body(a0_ref, a1_ref, y_ref, dinv_ref, b_ref, wl_ref, bl_ref,
                h_ref, o_ref):
    h2 = jax.nn.relu((a0_ref[...] + a1_ref[...] + y_ref[...]) * dinv_ref[...]
                     + b_ref[...])
    h_ref[...] = h2
    o_ref[...] = lax.dot_general(
        h2, wl_ref[...], (((1,), (0,)), ((), ())),
        preferred_element_type=jnp.float32) + bl_ref[...]


def _row_spec(cols):
    return pl.BlockSpec((BN, cols), lambda i: (i, 0))


def _full_spec(r, cols):
    return pl.BlockSpec((r, cols), lambda i: (0, 0))


def _gi_call(x2, W_ih, b_ih2):
    return pl.pallas_call(
        _gi_body,
        grid=(T * N // BN,),
        in_specs=[_row_spec(D), _full_spec(3 * H, D), _full_spec(1, 3 * H)],
        out_specs=_row_spec(3 * H),
        out_shape=jax.ShapeDtypeStruct((T * N, 3 * H), jnp.float32),
    )(x2, W_ih, b_ih2)


def _rsqrt_call(a0, a1):
    return pl.pallas_call(
        _rsqrt_body,
        grid=(N // BN,),
        in_specs=[_row_spec(H), _row_spec(H)],
        out_specs=_row_spec(H),
        out_shape=jax.ShapeDtypeStruct((N, H), jnp.float32),
    )(a0, a1)


def _gru_call(gi_t, h, W_hh, b_hh2, W0, dinv):
    return pl.pallas_call(
        _gru_body,
        grid=(N // BN,),
        in_specs=[_row_spec(3 * H), _row_spec(H), _full_spec(3 * H, H),
                  _full_spec(1, 3 * H), _full_spec(H, H), _row_spec(H)],
        out_specs=_row_spec(H),
        out_shape=jax.ShapeDtypeStruct((N, H), jnp.float32),
    )(gi_t, h, W_hh, b_hh2, W0, dinv)


def _post_call(a0, a1, y, dinv, b2, W):
    return pl.pallas_call(
        _post_body,
        grid=(N // BN,),
        in_specs=[_row_spec(H), _row_spec(H), _row_spec(H), _row_spec(H),
                  _full_spec(1, H), _full_spec(H, H)],
        out_specs=_row_spec(H),
        out_shape=jax.ShapeDtypeStruct((N, H), jnp.float32),
    )(a0, a1, y, dinv, b2, W)


def _final_call(a0, a1, y, dinv, b2, WlT, bl2):
    return pl.pallas_call(
        _final_body,
        grid=(N // BN,),
        in_specs=[_row_spec(H), _row_spec(H), _row_spec(H), _row_spec(H),
                  _full_spec(1, H), _full_spec(H, 1), _full_spec(1, 1)],
        out_specs=[_row_spec(H), _row_spec(1)],
        out_shape=[jax.ShapeDtypeStruct((N, H), jnp.float32),
                   jax.ShapeDtypeStruct((N, 1), jnp.float32)],
    )(a0, a1, y, dinv, b2, WlT, bl2)


def kernel(x, edge_index, edge_weight, W_ih, b_ih, W_hh, b_hh, W_conv, b_conv,
           W_lin, b_lin):
    src = edge_index[0]
    dst = edge_index[1]

    pad = NW * EPW - E
    src_p = jnp.pad(src, (0, pad)).reshape(NW, NH, NCH_H, CH)
    src_p = jnp.pad(src_p, ((0, 0), (0, 0), (0, 1), (0, 0)))
    dst_p = jnp.pad(dst, (0, pad)).reshape(NW, NH, NCH_H, CH)
    w_p = jnp.pad(edge_weight, (0, pad)).reshape(NW, NH, EPH)
    zeros = jnp.zeros((RPT, H), jnp.float32)

    mp = functools.partial(_mp_call, src_p=src_p, dst_p=dst_p, w_p=w_p,
                           zeros=zeros)

    acc_deg = mp(jnp.ones((N, H), jnp.float32))
    dinv = _rsqrt_call(acc_deg[0, :N], acc_deg[1, :N])

    gi_all = _gi_call(x.reshape(T * N, D), W_ih, b_ih.reshape(1, 3 * H))
    b_hh2 = b_hh.reshape(1, 3 * H)
    b_conv2 = [b_conv[l].reshape(1, H) for l in range(L)]
    WlT = W_lin.reshape(H, 1)
    bl2 = b_lin.reshape(1, 1)

    h = jnp.zeros((N, H), jnp.float32)
    outs = []
    for t in range(T):
        y0 = _gru_call(gi_all[t * N:(t + 1) * N], h, W_hh, b_hh2,
                       W_conv[0], dinv)
        a = mp(y0)
        y1 = _post_call(a[0, :N], a[1, :N], y0, dinv, b_conv2[0], W_conv[1])
        a = mp(y1)
        h, out_t = _final_call(a[0, :N], a[1, :N], y1, dinv, b_conv2[1], WlT, bl2)
        outs.append(out_t[None])
    return jnp.concatenate(outs, axis=0)


# speedup vs baseline: 1.4890x; 1.4890x over previous
import functools

import jax
import jax.numpy as jnp
from jax import lax
from jax.experimental import pallas as pl
from jax.experimental.pallas import tpu as pltpu
from jax.experimental.pallas import tpu_sc as plsc

T, N, E, D, H, L = 4, 10000, 320000, 128, 128, 2

NC, NS = 2, 16
NW = NC * NS
CH = 128
NCHUNK = 79
EPW = NCHUNK * CH
NP = 10240
RPT = NP // NS

BN = 1000


def _mp_body(y_hbm, src_hbm, dst_hbm, w_hbm, zeros_hbm, out_hbm,
             src_v, dst_v, w_v, rows_v, acc_sh, sem):
    c = lax.axis_index("c")
    s = lax.axis_index("s")
    wid = s * NC + c

    pltpu.sync_copy(zeros_hbm, acc_sh.at[pl.ds(s * RPT, RPT)])
    pltpu.sync_copy(src_hbm.at[wid], src_v)
    pltpu.sync_copy(dst_hbm.at[wid], dst_v)
    pltpu.sync_copy(w_hbm.at[wid], w_v)
    plsc.subcore_barrier()

    def chunk(j, carry):
        pltpu.async_copy(y_hbm.at[src_v.at[j]], rows_v, sem).wait()

        def scale(e, carry2):
            wspl = plsc.load_gather(
                w_v, [jnp.full((16,), j * CH + e, jnp.int32)])
            for v in range(8):
                rows_v[e, pl.ds(v * 16, 16)] = rows_v[e, pl.ds(v * 16, 16)] * wspl
            return carry2

        lax.fori_loop(0, CH, scale, 0)
        pltpu.sync_copy(rows_v, acc_sh.at[dst_v.at[j]], add=True)
        return carry

    lax.fori_loop(0, NCHUNK, chunk, 0)
    plsc.subcore_barrier()
    pltpu.sync_copy(acc_sh.at[pl.ds(s * RPT, RPT)],
                    out_hbm.at[c].at[pl.ds(s * RPT, RPT)])


def _mp_call(y, src_p, dst_p, w_p, zeros):
    mesh = plsc.VectorSubcoreMesh(core_axis_name="c", subcore_axis_name="s")
    return pl.kernel(
        _mp_body,
        out_type=jax.ShapeDtypeStruct((NC, NP, H), jnp.float32),
        mesh=mesh,
        compiler_params=pltpu.CompilerParams(needs_layout_passes=False),
        scratch_types=[
            pltpu.VMEM((NCHUNK, CH), jnp.int32),
            pltpu.VMEM((NCHUNK, CH), jnp.int32),
            pltpu.VMEM((EPW,), jnp.float32),
            pltpu.VMEM((CH, H), jnp.float32),
            pltpu.VMEM_SHARED((NP, H), jnp.float32),
            pltpu.SemaphoreType.DMA,
        ],
    )(y, src_p, dst_p, w_p, zeros)


def _gi_body(x_ref, w_ref, b_ref, o_ref):
    o_ref[...] = lax.dot_general(
        x_ref[...], w_ref[...], (((1,), (1,)), ((), ())),
        preferred_element_type=jnp.float32) + b_ref[...]


def _rsqrt_body(a0_ref, a1_ref, o_ref):
    o_ref[...] = lax.rsqrt(1.0 + a0_ref[...] + a1_ref[...])


def _gru_body(gi_ref, h_ref, whh_ref, bhh_ref, w0_ref, dinv_ref, y0_ref):
    h = h_ref[...]
    gh = lax.dot_general(h, whh_ref[...], (((1,), (1,)), ((), ())),
                         preferred_element_type=jnp.float32) + bhh_ref[...]
    gi = gi_ref[...]
    r = jax.nn.sigmoid(gi[:, :H] + gh[:, :H])
    z = jax.nn.sigmoid(gi[:, H:2 * H] + gh[:, H:2 * H])
    nc = jnp.tanh(gi[:, 2 * H:] + r * gh[:, 2 * H:])
    hn = (1.0 - z) * nc + z * h
    y0_ref[...] = lax.dot_general(
        hn, w0_ref[...], (((1,), (0,)), ((), ())),
        preferred_element_type=jnp.float32) * dinv_ref[...]


def _post_body(a0_ref, a1_ref, y_ref, dinv_ref, b_ref, w_ref, o_ref):
    dinv = dinv_ref[...]
    h1 = jax.nn.relu((a0_ref[...] + a1_ref[...] + y_ref[...]) * dinv + b_ref[...])
    o_ref[...] = lax.dot_general(
        h1, w_ref[...], (((1,), (0,)), ((), ())),
        preferred_element_type=jnp.float32) * dinv


def _final_body(a0_ref, a1_ref, y_ref, dinv_ref, b_ref, wl_ref, bl_ref,
                h_ref, o_ref):
    h2 = jax.nn.relu((a0_ref[...] + a1_ref[...] + y_ref[...]) * dinv_ref[...]
                     + b_ref[...])
    h_ref[...] = h2
    o_ref[...] = lax.dot_general(
        h2, wl_ref[...], (((1,), (0,)), ((), ())),
        preferred_element_type=jnp.float32) + bl_ref[...]


def _row_spec(cols):
    return pl.BlockSpec((BN, cols), lambda i: (i, 0))


def _full_spec(r, cols):
    return pl.BlockSpec((r, cols), lambda i: (0, 0))


def _gi_call(x2, W_ih, b_ih2):
    return pl.pallas_call(
        _gi_body,
        grid=(T * N // BN,),
        in_specs=[_row_spec(D), _full_spec(3 * H, D), _full_spec(1, 3 * H)],
        out_specs=_row_spec(3 * H),
        out_shape=jax.ShapeDtypeStruct((T * N, 3 * H), jnp.float32),
    )(x2, W_ih, b_ih2)


def _rsqrt_call(a0, a1):
    return pl.pallas_call(
        _rsqrt_body,
        grid=(N // BN,),
        in_specs=[_row_spec(H), _row_spec(H)],
        out_specs=_row_spec(H),
        out_shape=jax.ShapeDtypeStruct((N, H), jnp.float32),
    )(a0, a1)


def _gru_call(gi_t, h, W_hh, b_hh2, W0, dinv):
    return pl.pallas_call(
        _gru_body,
        grid=(N // BN,),
        in_specs=[_row_spec(3 * H), _row_spec(H), _full_spec(3 * H, H),
                  _full_spec(1, 3 * H), _full_spec(H, H), _row_spec(H)],
        out_specs=_row_spec(H),
        out_shape=jax.ShapeDtypeStruct((N, H), jnp.float32),
    )(gi_t, h, W_hh, b_hh2, W0, dinv)


def _post_call(a0, a1, y, dinv, b2, W):
    return pl.pallas_call(
        _post_body,
        grid=(N // BN,),
        in_specs=[_row_spec(H), _row_spec(H), _row_spec(H), _row_spec(H),
                  _full_spec(1, H), _full_spec(H, H)],
        out_specs=_row_spec(H),
        out_shape=jax.ShapeDtypeStruct((N, H), jnp.float32),
    )(a0, a1, y, dinv, b2, W)


def _final_call(a0, a1, y, dinv, b2, WlT, bl2):
    return pl.pallas_call(
        _final_body,
        grid=(N // BN,),
        in_specs=[_row_spec(H), _row_spec(H), _row_spec(H), _row_spec(H),
                  _full_spec(1, H), _full_spec(H, 1), _full_spec(1, 1)],
        out_specs=[_row_spec(H), _row_spec(1)],
        out_shape=[jax.ShapeDtypeStruct((N, H), jnp.float32),
                   jax.ShapeDtypeStruct((N, 1), jnp.float32)],
    )(a0, a1, y, dinv, b2, WlT, bl2)


def kernel(x, edge_index, edge_weight, W_ih, b_ih, W_hh, b_hh, W_conv, b_conv,
           W_lin, b_lin):
    src = edge_index[0]
    dst = edge_index[1]

    pad = NW * EPW - E
    src_p = jnp.pad(src, (0, pad)).reshape(NW, NCHUNK, CH)
    dst_p = jnp.pad(dst, (0, pad)).reshape(NW, NCHUNK, CH)
    w_p = jnp.pad(edge_weight, (0, pad)).reshape(NW, EPW)
    zeros = jnp.zeros((RPT, H), jnp.float32)

    mp = functools.partial(_mp_call, src_p=src_p, dst_p=dst_p, w_p=w_p,
                           zeros=zeros)

    acc_deg = mp(jnp.ones((N, H), jnp.float32))
    dinv = _rsqrt_call(acc_deg[0, :N], acc_deg[1, :N])

    gi_all = _gi_call(x.reshape(T * N, D), W_ih, b_ih.reshape(1, 3 * H))
    b_hh2 = b_hh.reshape(1, 3 * H)
    b_conv2 = [b_conv[l].reshape(1, H) for l in range(L)]
    WlT = W_lin.reshape(H, 1)
    bl2 = b_lin.reshape(1, 1)

    h = jnp.zeros((N, H), jnp.float32)
    outs = []
    for t in range(T):
        y0 = _gru_call(gi_all[t * N:(t + 1) * N], h, W_hh, b_hh2,
                       W_conv[0], dinv)
        a = mp(y0)
        y1 = _post_call(a[0, :N], a[1, :N], y0, dinv, b_conv2[0], W_conv[1])
        a = mp(y1)
        h, out_t = _final_call(a[0, :N], a[1, :N], y1, dinv, b_conv2[1], WlT, bl2)
        outs.append(out_t[None])
    return jnp.concatenate(outs, axis=0)


# speedup vs baseline: 1.5906x; 1.0682x over previous
import functools

import jax
import jax.numpy as jnp
from jax import lax
from jax.experimental import pallas as pl
from jax.experimental.pallas import tpu as pltpu
from jax.experimental.pallas import tpu_sc as plsc

T, N, E, D, H, L = 4, 10000, 320000, 128, 128, 2

NC, NS = 2, 16
NW = NC * NS
CH = 128
NCHUNK = 79
EPW = NCHUNK * CH
NP = 10240
RPT = NP // NS

BN = 1000


def _make_mp_body(W, fill_only=False):
    def _mp_body(y_hbm, src_hbm, dst_hbm, w_hbm, zeros_hbm, out_hbm,
                 src_v, dst_v, w_v, rows_v, acc_sh, sem):
        c = lax.axis_index("c")
        s = lax.axis_index("s")
        wid = s * NC + c

        pltpu.sync_copy(zeros_hbm, acc_sh.at[pl.ds(s * RPT, RPT)])
        pltpu.sync_copy(src_hbm.at[wid], src_v)
        pltpu.sync_copy(dst_hbm.at[wid], dst_v)
        pltpu.sync_copy(w_hbm.at[wid], w_v)
        plsc.subcore_barrier()

        def chunk(j, carry):
            if not fill_only:
                pltpu.async_copy(y_hbm.at[src_v.at[j]], rows_v, sem).wait()

            def scale(e, carry2):
                wspl = plsc.load_gather(
                    w_v, [jnp.full((16,), j * CH + e, jnp.int32)])
                for v in range(W // 16):
                    if fill_only:
                        rows_v[e, pl.ds(v * 16, 16)] = wspl
                    else:
                        rows_v[e, pl.ds(v * 16, 16)] = (
                            rows_v[e, pl.ds(v * 16, 16)] * wspl)
                return carry2

            lax.fori_loop(0, CH, scale, 0)
            pltpu.sync_copy(rows_v, acc_sh.at[dst_v.at[j]], add=True)
            return carry

        lax.fori_loop(0, NCHUNK, chunk, 0)
        plsc.subcore_barrier()
        pltpu.sync_copy(acc_sh.at[pl.ds(s * RPT, RPT)],
                        out_hbm.at[c].at[pl.ds(s * RPT, RPT)])

    return _mp_body


def _mp_call(y, src_p, dst_p, w_p, zeros, W=H, fill_only=False):
    mesh = plsc.VectorSubcoreMesh(core_axis_name="c", subcore_axis_name="s")
    return pl.kernel(
        _make_mp_body(W, fill_only),
        out_type=jax.ShapeDtypeStruct((NC, NP, W), jnp.float32),
        mesh=mesh,
        compiler_params=pltpu.CompilerParams(needs_layout_passes=False),
        scratch_types=[
            pltpu.VMEM((NCHUNK, CH), jnp.int32),
            pltpu.VMEM((NCHUNK, CH), jnp.int32),
            pltpu.VMEM((EPW,), jnp.float32),
            pltpu.VMEM((CH, W), jnp.float32),
            pltpu.VMEM_SHARED((NP, W), jnp.float32),
            pltpu.SemaphoreType.DMA,
        ],
    )(y, src_p, dst_p, w_p, zeros)


def _gi_body(x_ref, w_ref, b_ref, o_ref):
    o_ref[...] = lax.dot_general(
        x_ref[...], w_ref[...], (((1,), (1,)), ((), ())),
        preferred_element_type=jnp.float32) + b_ref[...]


def _rsqrt_body(a0_ref, a1_ref, o_ref):
    d = lax.rsqrt(1.0 + a0_ref[...] + a1_ref[...])
    o_ref[...] = jnp.broadcast_to(d[:, :1], (BN, H))


def _gru_body(gi_ref, h_ref, whh_ref, bhh_ref, w0_ref, dinv_ref, y0_ref):
    h = h_ref[...]
    gh = lax.dot_general(h, whh_ref[...], (((1,), (1,)), ((), ())),
                         preferred_element_type=jnp.float32) + bhh_ref[...]
    gi = gi_ref[...]
    r = jax.nn.sigmoid(gi[:, :H] + gh[:, :H])
    z = jax.nn.sigmoid(gi[:, H:2 * H] + gh[:, H:2 * H])
    nc = jnp.tanh(gi[:, 2 * H:] + r * gh[:, 2 * H:])
    hn = (1.0 - z) * nc + z * h
    y0_ref[...] = lax.dot_general(
        hn, w0_ref[...], (((1,), (0,)), ((), ())),
        preferred_element_type=jnp.float32) * dinv_ref[...]


def _post_body(a0_ref, a1_ref, y_ref, dinv_ref, b_ref, w_ref, o_ref):
    dinv = dinv_ref[...]
    h1 = jax.nn.relu((a0_ref[...] + a1_ref[...] + y_ref[...]) * dinv + b_ref[...])
    o_ref[...] = lax.dot_general(
        h1, w_ref[...], (((1,), (0,)), ((), ())),
        preferred_element_type=jnp.float32) * dinv


def _final_body(a0_ref, a1_ref, y_ref, dinv_ref, b_ref, wl_ref, bl_ref,
                h_ref, o_ref):
    h2 = jax.nn.relu((a0_ref[...] + a1_ref[...] + y_ref[...]) * dinv_ref[...]
                     + b_ref[...])
    h_ref[...] = h2
    o_ref[...] = lax.dot_general(
        h2, wl_ref[...], (((1,), (0,)), ((), ())),
        preferred_element_type=jnp.float32) + bl_ref[...]


def _row_spec(cols):
    return pl.BlockSpec((BN, cols), lambda i: (i, 0))


def _full_spec(r, cols):
    return pl.BlockSpec((r, cols), lambda i: (0, 0))


def _gi_call(x2, W_ih, b_ih2):
    return pl.pallas_call(
        _gi_body,
        grid=(T * N // BN,),
        in_specs=[_row_spec(D), _full_spec(3 * H, D), _full_spec(1, 3 * H)],
        out_specs=_row_spec(3 * H),
        out_shape=jax.ShapeDtypeStruct((T * N, 3 * H), jnp.float32),
    )(x2, W_ih, b_ih2)


def _rsqrt_call(a0, a1):
    return pl.pallas_call(
        _rsqrt_body,
        grid=(N // BN,),
        in_specs=[_row_spec(H), _row_spec(H)],
        out_specs=_row_spec(H),
        out_shape=jax.ShapeDtypeStruct((N, H), jnp.float32),
    )(a0, a1)


def _gru_call(gi_t, h, W_hh, b_hh2, W0, dinv):
    return pl.pallas_call(
        _gru_body,
        grid=(N // BN,),
        in_specs=[_row_spec(3 * H), _row_spec(H), _full_spec(3 * H, H),
                  _full_spec(1, 3 * H), _full_spec(H, H), _row_spec(H)],
        out_specs=_row_spec(H),
        out_shape=jax.ShapeDtypeStruct((N, H), jnp.float32),
    )(gi_t, h, W_hh, b_hh2, W0, dinv)


def _post_call(a0, a1, y, dinv, b2, W):
    return pl.pallas_call(
        _post_body,
        grid=(N // BN,),
        in_specs=[_row_spec(H), _row_spec(H), _row_spec(H), _row_spec(H),
                  _full_spec(1, H), _full_spec(H, H)],
        out_specs=_row_spec(H),
        out_shape=jax.ShapeDtypeStruct((N, H), jnp.float32),
    )(a0, a1, y, dinv, b2, W)


def _final_call(a0, a1, y, dinv, b2, WlT, bl2):
    return pl.pallas_call(
        _final_body,
        grid=(N // BN,),
        in_specs=[_row_spec(H), _row_spec(H), _row_spec(H), _row_spec(H),
                  _full_spec(1, H), _full_spec(H, 1), _full_spec(1, 1)],
        out_specs=[_row_spec(H), _row_spec(1)],
        out_shape=[jax.ShapeDtypeStruct((N, H), jnp.float32),
                   jax.ShapeDtypeStruct((N, 1), jnp.float32)],
    )(a0, a1, y, dinv, b2, WlT, bl2)


def kernel(x, edge_index, edge_weight, W_ih, b_ih, W_hh, b_hh, W_conv, b_conv,
           W_lin, b_lin):
    src = edge_index[0]
    dst = edge_index[1]

    pad = NW * EPW - E
    src_p = jnp.pad(src, (0, pad)).reshape(NW, NCHUNK, CH)
    dst_p = jnp.pad(dst, (0, pad)).reshape(NW, NCHUNK, CH)
    w_p = jnp.pad(edge_weight, (0, pad)).reshape(NW, EPW)
    zeros = jnp.zeros((RPT, H), jnp.float32)

    mp = functools.partial(_mp_call, src_p=src_p, dst_p=dst_p, w_p=w_p,
                           zeros=zeros)

    acc_deg = _mp_call(jnp.ones((N, H), jnp.float32), src_p, dst_p, w_p,
                       zeros, fill_only=True)
    dinv = _rsqrt_call(acc_deg[0, :N], acc_deg[1, :N])

    gi_all = _gi_call(x.reshape(T * N, D), W_ih, b_ih.reshape(1, 3 * H))
    b_hh2 = b_hh.reshape(1, 3 * H)
    b_conv2 = [b_conv[l].reshape(1, H) for l in range(L)]
    WlT = W_lin.reshape(H, 1)
    bl2 = b_lin.reshape(1, 1)

    h = jnp.zeros((N, H), jnp.float32)
    outs = []
    for t in range(T):
        y0 = _gru_call(gi_all[t * N:(t + 1) * N], h, W_hh, b_hh2,
                       W_conv[0], dinv)
        a = mp(y0)
        y1 = _post_call(a[0, :N], a[1, :N], y0, dinv, b_conv2[0], W_conv[1])
        a = mp(y1)
        h, out_t = _final_call(a[0, :N], a[1, :N], y1, dinv, b_conv2[1], WlT, bl2)
        outs.append(out_t[None])
    return jnp.concatenate(outs, axis=0)
